# Initial kernel scaffold; baseline (speedup 1.0000x reference)
import jax
import jax.numpy as jnp
from jax.experimental import pallas as pl


def kernel(meanings, src):
    raise NotImplementedError("write your pallas kernel here")


# speedup vs baseline: 3.8470x; 3.8470x over previous
import jax
import jax.numpy as jnp
from jax.experimental import pallas as pl

UTT_LEN = 20
N = 4096
VOCAB1 = 129


def _zero_like(i):
    return i * 0


def _onehot_fill(src_ref, o_ref):
    s = src_ref[0, 0, :]
    lane = jax.lax.broadcasted_iota(jnp.int32, (N, VOCAB1), 1)
    o_ref[0] = jnp.where(lane == 0, s[:, None], jnp.float32(0.0))


def kernel(meanings, src):
    del meanings
    src3 = src.astype(jnp.float32).reshape(UTT_LEN, 1, N)
    return pl.pallas_call(
        _onehot_fill,
        grid=(UTT_LEN,),
        in_specs=[pl.BlockSpec((1, 1, N), lambda i: (i, _zero_like(i), _zero_like(i)))],
        out_specs=pl.BlockSpec((1, N, VOCAB1), lambda i: (i, _zero_like(i), _zero_like(i))),
        out_shape=jax.ShapeDtypeStruct((UTT_LEN, N, VOCAB1), jnp.float32),
    )(src3)
